# baseline (speedup 1.0000x reference)
import functools

import jax
import jax.numpy as jnp
from jax import lax
from jax.experimental import pallas as pl
from jax.experimental.pallas import tpu as pltpu
from jax.experimental.pallas import tpu_sc as plsc

_LANES = 16


def _perm(vec, idx):
    return lax.gather(
        vec,
        idx.reshape(_LANES, 1),
        lax.GatherDimensionNumbers(
            offset_dims=(), collapsed_slice_dims=(0,), start_index_map=(0,)
        ),
        (1,),
        mode=lax.GatherScatterMode.PROMISE_IN_BOUNDS,
    )


def _bcast_lane(vec, k):
    return _perm(vec, jnp.full((_LANES,), k, jnp.int32))


def _allreduce_sum(vec, lane):
    for shift in (1, 2, 4, 8):
        vec = vec + _perm(vec, jnp.bitwise_xor(lane, shift))
    return vec


def _rsqrt_newton(x):
    i = plsc.bitcast(x, jnp.int32)
    y = plsc.bitcast(jnp.int32(0x5F3759DF) - (i >> 1), jnp.float32)
    for _ in range(3):
        y = y * (1.5 - 0.5 * x * y * y)
    return y


def kernel(indices, table, W, b):
    batch = indices.shape[0]
    vocab, d = table.shape
    info = plsc.get_sparse_core_info()
    nc, ns = info.num_cores, info.num_subcores
    nw = nc * ns
    bpw = batch // nw
    ngroups = bpw // _LANES

    tab_flat = table.reshape(vocab * d)
    w_flat = W.reshape(d * d)
    idx2 = indices.reshape(nw, bpw)

    mesh = plsc.VectorSubcoreMesh(core_axis_name="c", subcore_axis_name="s")

    @functools.partial(
        pl.kernel,
        mesh=mesh,
        compiler_params=pltpu.CompilerParams(
            use_tc_tiling_on_sc=False, needs_layout_passes=False
        ),
        out_type=jax.ShapeDtypeStruct((batch * d,), jnp.float32),
        scratch_types=[
            pltpu.VMEM((vocab * d,), jnp.float32),
            pltpu.VMEM((d * d,), jnp.float32),
            pltpu.VMEM((d,), jnp.float32),
            pltpu.VMEM((vocab * d,), jnp.float32),
            pltpu.VMEM((bpw,), jnp.int32),
            pltpu.VMEM((bpw * d,), jnp.float32),
        ],
    )
    def _enc(tab_hbm, w_hbm, b_hbm, idx_hbm, out_hbm,
             tabin_v, w_v, b_v, tab_v, idx_w, rows_v):
        wid = lax.axis_index("s") * nc + lax.axis_index("c")
        pltpu.sync_copy(tab_hbm, tabin_v)
        pltpu.sync_copy(w_hbm, w_v)
        pltpu.sync_copy(b_hbm, b_v)
        pltpu.sync_copy(idx_hbm.at[wid], idx_w)

        lane = lax.iota(jnp.int32, _LANES)
        w_rows = [w_v[pl.ds(dd * d, d)] for dd in range(d)]
        wcols = []
        for k in range(d):
            col = jnp.zeros((_LANES,), jnp.float32)
            for dd in range(d):
                col = jnp.where(lane == dd, _bcast_lane(w_rows[dd], k), col)
            wcols.append(col)
        bvec = b_v[...]

        for v in range(vocab):
            row = tabin_v[pl.ds(v * d, d)]
            acc = bvec
            for k in range(d):
                acc = acc + _bcast_lane(row, k) * wcols[k]
            tot = jnp.maximum(_allreduce_sum(acc * acc, lane), 1e-24)
            acc = acc * _rsqrt_newton(tot)
            tab_v[pl.ds(v * d, d)] = acc

        lane_row = lane * d

        def group_body(g, carry):
            base = pl.multiple_of(g * _LANES, _LANES)
            ridx = idx_w[pl.ds(base, _LANES)]
            src_base = ridx * d
            dst_base = g * (_LANES * d) + lane_row
            for dcol in range(d):
                col = plsc.load_gather(tab_v, [src_base + dcol])
                plsc.store_scatter(rows_v, [dst_base + dcol], col)
            return carry

        lax.fori_loop(0, ngroups, group_body, 0, unroll=8)
        pltpu.sync_copy(rows_v, out_hbm.at[pl.ds(wid * bpw * d, bpw * d)])

    return _enc(tab_flat, w_flat, b, idx2).reshape(batch, d)


# speedup vs baseline: 1.0160x; 1.0160x over previous
import functools

import jax
import jax.numpy as jnp
from jax import lax
from jax.experimental import pallas as pl
from jax.experimental.pallas import tpu as pltpu
from jax.experimental.pallas import tpu_sc as plsc

_LANES = 16


def _perm(vec, idx):
    return lax.gather(
        vec,
        idx.reshape(_LANES, 1),
        lax.GatherDimensionNumbers(
            offset_dims=(), collapsed_slice_dims=(0,), start_index_map=(0,)
        ),
        (1,),
        mode=lax.GatherScatterMode.PROMISE_IN_BOUNDS,
    )


def _bcast_lane(vec, k):
    return _perm(vec, jnp.full((_LANES,), k, jnp.int32))


def _allreduce_sum(vec, lane):
    for shift in (1, 2, 4, 8):
        vec = vec + _perm(vec, jnp.bitwise_xor(lane, shift))
    return vec


def _rsqrt_newton(x):
    i = plsc.bitcast(x, jnp.int32)
    y = plsc.bitcast(jnp.int32(0x5F3759DF) - (i >> 1), jnp.float32)
    for _ in range(3):
        y = y * (1.5 - 0.5 * x * y * y)
    return y


def kernel(indices, table, W, b):
    batch = indices.shape[0]
    vocab, d = table.shape
    info = plsc.get_sparse_core_info()
    nc, ns = info.num_cores, info.num_subcores
    nw = nc * ns
    bpw = batch // nw
    ngroups = bpw // _LANES

    tab_flat = table.reshape(vocab * d)
    w_flat = W.reshape(d * d)

    mesh = plsc.VectorSubcoreMesh(core_axis_name="c", subcore_axis_name="s")

    @functools.partial(
        pl.kernel,
        mesh=mesh,
        compiler_params=pltpu.CompilerParams(needs_layout_passes=False),
        out_type=jax.ShapeDtypeStruct((batch, d), jnp.float32),
        scratch_types=[
            pltpu.VMEM((vocab * d,), jnp.float32),
            pltpu.VMEM((d * d,), jnp.float32),
            pltpu.VMEM((d,), jnp.float32),
            pltpu.VMEM((vocab * d,), jnp.float32),
            pltpu.VMEM((bpw,), jnp.int32),
            pltpu.VMEM((bpw, d), jnp.float32),
        ],
    )
    def _enc(tab_hbm, w_hbm, b_hbm, idx_hbm, out_hbm,
             tabin_v, w_v, b_v, tab_v, idx_w, rows_v):
        wid = lax.axis_index("s") * nc + lax.axis_index("c")
        pltpu.sync_copy(tab_hbm, tabin_v)
        pltpu.sync_copy(w_hbm, w_v)
        pltpu.sync_copy(b_hbm, b_v)
        pltpu.sync_copy(idx_hbm.at[pl.ds(wid * bpw, bpw)], idx_w)

        lane = lax.iota(jnp.int32, _LANES)
        w_rows = [w_v[pl.ds(dd * d, d)] for dd in range(d)]
        wcols = []
        for k in range(d):
            col = jnp.zeros((_LANES,), jnp.float32)
            for dd in range(d):
                col = jnp.where(lane == dd, _bcast_lane(w_rows[dd], k), col)
            wcols.append(col)
        bvec = b_v[...]

        for v in range(vocab):
            row = tabin_v[pl.ds(v * d, d)]
            acc = bvec
            for k in range(d):
                acc = acc + _bcast_lane(row, k) * wcols[k]
            tot = jnp.maximum(_allreduce_sum(acc * acc, lane), 1e-24)
            acc = acc * _rsqrt_newton(tot)
            tab_v[pl.ds(v * d, d)] = acc

        def group_body(g, carry):
            base = pl.multiple_of(g * _LANES, _LANES)
            ridx = idx_w[pl.ds(base, _LANES)]
            src_base = ridx * d
            dst_row = base + lane
            for dcol in range(d):
                col = plsc.load_gather(tab_v, [src_base + dcol])
                plsc.store_scatter(
                    rows_v, [dst_row, jnp.full((_LANES,), dcol, jnp.int32)], col
                )
            return carry

        lax.fori_loop(0, ngroups, group_body, 0, unroll=8)
        pltpu.sync_copy(rows_v, out_hbm.at[pl.ds(wid * bpw, bpw)])

    return _enc(tab_flat, w_flat, b, indices)


# speedup vs baseline: 1.1380x; 1.1201x over previous
import functools

import jax
import jax.numpy as jnp
from jax import lax
from jax.experimental import pallas as pl
from jax.experimental.pallas import tpu as pltpu
from jax.experimental.pallas import tpu_sc as plsc

_LANES = 16


def _perm(vec, idx):
    return lax.gather(
        vec,
        idx.reshape(_LANES, 1),
        lax.GatherDimensionNumbers(
            offset_dims=(), collapsed_slice_dims=(0,), start_index_map=(0,)
        ),
        (1,),
        mode=lax.GatherScatterMode.PROMISE_IN_BOUNDS,
    )


def _bcast_lane(vec, k):
    return _perm(vec, jnp.full((_LANES,), k, jnp.int32))


def _allreduce_sum(vec, lane):
    for shift in (1, 2, 4, 8):
        vec = vec + _perm(vec, jnp.bitwise_xor(lane, shift))
    return vec


def _rsqrt_newton(x):
    i = plsc.bitcast(x, jnp.int32)
    y = plsc.bitcast(jnp.int32(0x5F3759DF) - (i >> 1), jnp.float32)
    for _ in range(3):
        y = y * (1.5 - 0.5 * x * y * y)
    return y


def kernel(indices, table, W, b):
    batch = indices.shape[0]
    vocab, d = table.shape
    info = plsc.get_sparse_core_info()
    nc, ns = info.num_cores, info.num_subcores
    nw = nc * ns
    bpw = batch // nw
    ngroups = bpw // _LANES

    tab_flat = table.reshape(vocab * d)
    w_flat = W.reshape(d * d)

    mesh = plsc.VectorSubcoreMesh(core_axis_name="c", subcore_axis_name="s")

    @functools.partial(
        pl.kernel,
        mesh=mesh,
        compiler_params=pltpu.CompilerParams(needs_layout_passes=False),
        out_type=jax.ShapeDtypeStruct((batch, d), jnp.float32),
        scratch_types=[
            pltpu.VMEM((vocab * d,), jnp.float32),
            pltpu.VMEM((d * d,), jnp.float32),
            pltpu.VMEM((d,), jnp.float32),
            pltpu.VMEM((vocab * d,), jnp.float32),
            pltpu.VMEM((bpw,), jnp.int32),
            pltpu.VMEM((bpw, d), jnp.float32),
        ],
    )
    def _enc(tab_hbm, w_hbm, b_hbm, idx_hbm, out_hbm,
             tabin_v, w_v, b_v, tab_v, idx_w, rows_v):
        wid = lax.axis_index("s") * nc + lax.axis_index("c")
        pltpu.sync_copy(tab_hbm, tabin_v)
        pltpu.sync_copy(w_hbm, w_v)
        pltpu.sync_copy(b_hbm, b_v)
        pltpu.sync_copy(idx_hbm.at[pl.ds(wid * bpw, bpw)], idx_w)

        lane = lax.iota(jnp.int32, _LANES)
        w_rows = [w_v[pl.ds(dd * d, d)] for dd in range(d)]
        wcols = []
        for k in range(d):
            col = jnp.zeros((_LANES,), jnp.float32)
            for dd in range(d):
                col = jnp.where(lane == dd, _bcast_lane(w_rows[dd], k), col)
            wcols.append(col)
        bvec = b_v[...]

        @plsc.parallel_loop(0, vocab, 1, unroll=2)
        def _fuse_body(v):
            off = pl.multiple_of(v * d, d)
            row = tabin_v[pl.ds(off, d)]
            acc = bvec
            for k in range(d):
                acc = acc + _bcast_lane(row, k) * wcols[k]
            tot = jnp.maximum(_allreduce_sum(acc * acc, lane), 1e-24)
            tab_v[pl.ds(off, d)] = acc * _rsqrt_newton(tot)

        @plsc.parallel_loop(0, ngroups, 1, unroll=4)
        def _gather_body(g):
            base = pl.multiple_of(g * _LANES, _LANES)
            ridx = idx_w[pl.ds(base, _LANES)]
            src_base = ridx * d
            dst_row = base + lane
            cols = [plsc.load_gather(tab_v, [src_base + dcol]) for dcol in range(d)]
            for dcol in range(d):
                plsc.store_scatter(
                    rows_v,
                    [dst_row, jnp.full((_LANES,), dcol, jnp.int32)],
                    cols[dcol],
                )
        pltpu.sync_copy(rows_v, out_hbm.at[pl.ds(wid * bpw, bpw)])

    return _enc(tab_flat, w_flat, b, indices)
